# Initial kernel scaffold; baseline (speedup 1.0000x reference)
import functools

import jax
import jax.numpy as jnp
from jax import lax
from jax.experimental import pallas as pl
from jax.experimental.pallas import tpu as pltpu
from jax.experimental.pallas import tpu_sc as plsc

_N = 10000
_M = 2500
_E = 320000
_D = 128

_NC = 2
_NS = 16
_NW = _NC * _NS
_EPW = _E // _NW
_C = 80
_K = _EPW // _C


def _proj_body(vf_ref, dv2_ref, w1_ref, b1_ref, out_ref):
    wh = jnp.dot(vf_ref[...], w1_ref[...],
                 preferred_element_type=jnp.float32) + b1_ref[...]
    out_ref[...] = dv2_ref[...] * wh


def _proj(vfeat, dv2_col, w1, b1_row):
    blk = 1000
    grid = _N // blk
    return pl.pallas_call(
        _proj_body,
        grid=(grid,),
        in_specs=[
            pl.BlockSpec((blk, _D), lambda i: (i, 0)),
            pl.BlockSpec((blk, 1), lambda i: (i, 0)),
            pl.BlockSpec((_D, _D), lambda i: (0, 0)),
            pl.BlockSpec((1, _D), lambda i: (0, 0)),
        ],
        out_specs=pl.BlockSpec((blk, _D), lambda i: (i, 0)),
        out_shape=jax.ShapeDtypeStruct((_N, _D), jnp.float32),
    )(vfeat, dv2_col, w1, b1_row)


def _comb_body(p_ref, inv_ref, out_ref):
    out_ref[...] = inv_ref[...] * (p_ref[0] + p_ref[1])


def _comb(partials, inv_col):
    return pl.pallas_call(
        _comb_body,
        out_shape=jax.ShapeDtypeStruct((_M, _D), jnp.float32),
    )(partials, inv_col)


def _final_body(p_ref, dv2_ref, wc_ref, bc_ref, vout_ref, preds_ref):
    a = p_ref[0] + p_ref[1]
    h = jnp.maximum(dv2_ref[...] * a, 0.0)
    vout_ref[...] = h
    logits = jnp.dot(h, wc_ref[...],
                     preferred_element_type=jnp.float32) + bc_ref[...]
    m = jnp.max(logits, axis=-1, keepdims=True)
    e = jnp.exp(logits - m)
    preds_ref[...] = e / jnp.sum(e, axis=-1, keepdims=True)


def _final(partials, dv2_col, wc, bc_row):
    blk = 1000
    grid = _N // blk
    ncls = wc.shape[1]
    return pl.pallas_call(
        _final_body,
        grid=(grid,),
        in_specs=[
            pl.BlockSpec((2, blk, _D), lambda i: (0, i, 0)),
            pl.BlockSpec((blk, 1), lambda i: (i, 0)),
            pl.BlockSpec((_D, ncls), lambda i: (0, 0)),
            pl.BlockSpec((1, ncls), lambda i: (0, 0)),
        ],
        out_specs=[
            pl.BlockSpec((blk, _D), lambda i: (i, 0)),
            pl.BlockSpec((blk, ncls), lambda i: (i, 0)),
        ],
        out_shape=[
            jax.ShapeDtypeStruct((_N, _D), jnp.float32),
            jax.ShapeDtypeStruct((_N, ncls), jnp.float32),
        ],
    )(partials, dv2_col, wc, bc_row)


def _make_stage(table_rows, acc_rows):
    r = acc_rows // _NS
    rem = acc_rows - r * _NS
    mesh = plsc.VectorSubcoreMesh(core_axis_name="c", subcore_axis_name="s",
                                  num_cores=_NC, num_subcores=_NS)

    def body(table_hbm, gidx_hbm, sidx_hbm, zeros_hbm, out_hbm,
             gidx_v, sidx_v, rows_v, obuf_v, rbuf_v, acc_sh, sem):
        c = lax.axis_index("c")
        s = lax.axis_index("s")
        wid = c * _NS + s
        rowblk = wid * _K
        pltpu.sync_copy(gidx_hbm.at[pl.ds(rowblk, _K)], gidx_v)
        pltpu.sync_copy(sidx_hbm.at[pl.ds(rowblk, _K)], sidx_v)
        pltpu.sync_copy(zeros_hbm.at[pl.ds(0, r)], obuf_v)
        pltpu.sync_copy(obuf_v, acc_sh.at[pl.ds(s * r, r)])
        if rem:
            @pl.when(s == 0)
            def _():
                pltpu.sync_copy(zeros_hbm.at[pl.ds(0, rem)], rbuf_v)
                pltpu.sync_copy(rbuf_v, acc_sh.at[pl.ds(_NS * r, rem)])
        plsc.subcore_barrier()

        def chunk(j, carry):
            pltpu.async_copy(table_hbm.at[gidx_v.at[j]], rows_v, sem).wait()
            pltpu.sync_copy(rows_v, acc_sh.at[sidx_v.at[j]], add=True)
            return carry
        lax.fori_loop(0, _K, chunk, 0)

        plsc.subcore_barrier()
        pltpu.sync_copy(acc_sh.at[pl.ds(s * r, r)], obuf_v)
        pltpu.sync_copy(obuf_v, out_hbm.at[c, pl.ds(s * r, r)])
        if rem:
            @pl.when(s == 0)
            def _():
                pltpu.sync_copy(acc_sh.at[pl.ds(_NS * r, rem)], rbuf_v)
                pltpu.sync_copy(rbuf_v, out_hbm.at[c, pl.ds(_NS * r, rem)])

    kern = pl.kernel(
        body,
        out_type=jax.ShapeDtypeStruct((_NC, acc_rows, _D), jnp.float32),
        mesh=mesh,
        scratch_types=[
            pltpu.VMEM((_K, _C), jnp.int32),
            pltpu.VMEM((_K, _C), jnp.int32),
            pltpu.VMEM((_C, _D), jnp.float32),
            pltpu.VMEM((r, _D), jnp.float32),
            pltpu.VMEM((max(rem, 1), _D), jnp.float32),
            pltpu.VMEM_SHARED((acc_rows, _D), jnp.float32),
            pltpu.SemaphoreType.DMA,
        ],
    )
    return kern


_stage1 = _make_stage(_N, _M)
_stage2 = _make_stage(_M, _N)


def kernel(vfeat, DV2, invDE, W1, b1, Wc, bc, node_idx, edge_idx,
           first_layer, last_layer):
    nidx = node_idx.astype(jnp.int32).reshape(_NW * _K, _C)
    eidx = edge_idx.astype(jnp.int32).reshape(_NW * _K, _C)
    zeros = jnp.zeros((_N, _D), jnp.float32)
    dv2_col = DV2.reshape(_N, 1)

    x = _proj(vfeat, dv2_col, W1, b1.reshape(1, _D))
    p_edge = _stage1(x, nidx, eidx, zeros)
    g = _comb(p_edge, invDE.reshape(_M, 1))
    p_node = _stage2(g, eidx, nidx, zeros)
    vout, preds = _final(p_node, dv2_col, Wc, bc.reshape(1, -1))
    return (vout, preds)


# speedup vs baseline: 16.1152x; 16.1152x over previous
import functools

import jax
import jax.numpy as jnp
from jax import lax
from jax.experimental import pallas as pl
from jax.experimental.pallas import tpu as pltpu
from jax.experimental.pallas import tpu_sc as plsc

_N = 10000
_M = 2500
_E = 320000
_D = 128

_MP = 2560
_NP = 10240

_NC = 2
_NS = 16
_NW = _NC * _NS
_EPW = _E // _NW
_C = 80
_K = _EPW // _C


def _proj_body(vf_ref, dv2_ref, w1_ref, b1_ref, out_ref):
    wh = jnp.dot(vf_ref[...], w1_ref[...],
                 preferred_element_type=jnp.float32) + b1_ref[...]
    out_ref[...] = dv2_ref[...] * wh


def _proj(vfeat, dv2_col, w1, b1_row):
    blk = 1000
    grid = _N // blk
    return pl.pallas_call(
        _proj_body,
        grid=(grid,),
        in_specs=[
            pl.BlockSpec((blk, _D), lambda i: (i, 0)),
            pl.BlockSpec((blk, 1), lambda i: (i, 0)),
            pl.BlockSpec((_D, _D), lambda i: (0, 0)),
            pl.BlockSpec((1, _D), lambda i: (0, 0)),
        ],
        out_specs=pl.BlockSpec((blk, _D), lambda i: (i, 0)),
        out_shape=jax.ShapeDtypeStruct((_N, _D), jnp.float32),
    )(vfeat, dv2_col, w1, b1_row)


def _comb_body(p_ref, inv_ref, out_ref):
    out_ref[...] = inv_ref[...] * (p_ref[0] + p_ref[1])


def _comb(partials, inv_col):
    rows = partials.shape[1]
    return pl.pallas_call(
        _comb_body,
        out_shape=jax.ShapeDtypeStruct((rows, _D), jnp.float32),
    )(partials, inv_col)


def _final_body(p_ref, dv2_ref, wc_ref, bc_ref, vout_ref, preds_ref):
    a = p_ref[0] + p_ref[1]
    h = jnp.maximum(dv2_ref[...] * a, 0.0)
    vout_ref[...] = h
    logits = jnp.dot(h, wc_ref[...],
                     preferred_element_type=jnp.float32) + bc_ref[...]
    m = jnp.max(logits, axis=-1, keepdims=True)
    e = jnp.exp(logits - m)
    preds_ref[...] = e / jnp.sum(e, axis=-1, keepdims=True)


def _final(partials, dv2_col, wc, bc_row):
    blk = 1000
    grid = _N // blk
    ncls = wc.shape[1]
    return pl.pallas_call(
        _final_body,
        grid=(grid,),
        in_specs=[
            pl.BlockSpec((2, blk, _D), lambda i: (0, i, 0)),
            pl.BlockSpec((blk, 1), lambda i: (i, 0)),
            pl.BlockSpec((_D, ncls), lambda i: (0, 0)),
            pl.BlockSpec((1, ncls), lambda i: (0, 0)),
        ],
        out_specs=[
            pl.BlockSpec((blk, _D), lambda i: (i, 0)),
            pl.BlockSpec((blk, ncls), lambda i: (i, 0)),
        ],
        out_shape=[
            jax.ShapeDtypeStruct((_N, _D), jnp.float32),
            jax.ShapeDtypeStruct((_N, ncls), jnp.float32),
        ],
    )(partials, dv2_col, wc, bc_row)


def _make_stage(table_rows, acc_pad):
    r = acc_pad // _NS
    mesh = plsc.VectorSubcoreMesh(core_axis_name="c", subcore_axis_name="s",
                                  num_cores=_NC, num_subcores=_NS)

    cr = 32
    nck = r // cr
    assert r % cr == 0

    def body(table_hbm, gidx_hbm, sidx_hbm, zeros_hbm, out_hbm,
             gidx_v, sidx_v, rows_v, obuf_v, acc_sh, sem):
        c = lax.axis_index("c")
        s = lax.axis_index("s")
        wid = c * _NS + s
        pltpu.sync_copy(gidx_hbm.at[wid], gidx_v)
        pltpu.sync_copy(sidx_hbm.at[wid], sidx_v)
        pltpu.sync_copy(zeros_hbm.at[pl.ds(0, cr)], obuf_v)

        def zchunk(t, carry):
            pltpu.sync_copy(obuf_v, acc_sh.at[pl.ds(s * r + t * cr, cr)])
            return carry
        lax.fori_loop(0, nck, zchunk, 0)
        plsc.subcore_barrier()

        def chunk(j, carry):
            pltpu.async_copy(table_hbm.at[gidx_v.at[j]], rows_v, sem).wait()
            pltpu.sync_copy(rows_v, acc_sh.at[sidx_v.at[j]], add=True)
            return carry
        lax.fori_loop(0, _K, chunk, 0)

        plsc.subcore_barrier()

        def wchunk(t, carry):
            pltpu.sync_copy(acc_sh.at[pl.ds(s * r + t * cr, cr)], obuf_v)
            pltpu.sync_copy(obuf_v, out_hbm.at[c, pl.ds(s * r + t * cr, cr)])
            return carry
        lax.fori_loop(0, nck, wchunk, 0)

    kern = pl.kernel(
        body,
        out_type=jax.ShapeDtypeStruct((_NC, acc_pad, _D), jnp.float32),
        mesh=mesh,
        scratch_types=[
            pltpu.VMEM((_K, _C), jnp.int32),
            pltpu.VMEM((_K, _C), jnp.int32),
            pltpu.VMEM((_C, _D), jnp.float32),
            pltpu.VMEM((cr, _D), jnp.float32),
            pltpu.VMEM_SHARED((acc_pad, _D), jnp.float32),
            pltpu.SemaphoreType.DMA,
        ],
    )
    return kern


@functools.lru_cache(maxsize=None)
def _get_stage(table_rows, acc_rows):
    return _make_stage(table_rows, acc_rows)


def _stage1(x, gidx, sidx, zeros):
    return _get_stage(_N, _MP)(x, gidx, sidx, zeros)


def _stage2(g, gidx, sidx, zeros):
    return _get_stage(_MP, _NP)(g, gidx, sidx, zeros)


def kernel(vfeat, DV2, invDE, W1, b1, Wc, bc, node_idx, edge_idx,
           first_layer, last_layer):
    nidx = node_idx.astype(jnp.int32).reshape(_NW, _K, _C)
    eidx = edge_idx.astype(jnp.int32).reshape(_NW, _K, _C)
    zeros = jnp.zeros((_N, _D), jnp.float32)
    dv2_col = DV2.reshape(_N, 1)

    inv_col = jnp.zeros((_MP, 1), jnp.float32).at[:_M].set(
        invDE.reshape(_M, 1))

    x = _proj(vfeat, dv2_col, W1, b1.reshape(1, _D))
    p_edge = _stage1(x, nidx, eidx, zeros)
    g = _comb(p_edge, inv_col)
    p_node = _stage2(g, eidx, nidx, zeros)
    vout, preds = _final(p_node, dv2_col, Wc, bc.reshape(1, -1))
    return (vout, preds)
